# Initial kernel scaffold; baseline (speedup 1.0000x reference)
import jax
import jax.numpy as jnp
from jax.experimental import pallas as pl


def kernel(x, freqs_cos, freqs_sin, task_ids, n1w, n2w, Wq, Wk, Wv, Wo, qA, qB, kA, kB, vA, vB, gateW, gA, gB, temb, W1, b1, Wg, bg, Wve, bv):
    raise NotImplementedError("write your pallas kernel here")


# speedup vs baseline: 1.3370x; 1.3370x over previous
import functools

import jax
import jax.numpy as jnp
import numpy as np
from jax import lax
from jax.experimental import pallas as pl
from jax.experimental.pallas import tpu as pltpu
from jax.experimental.pallas import tpu_sc as plsc

S, D = 2048, 1024
NH, NKV, DK = 16, 4, 64
E, TOPK, R, NT = 8, 2, 16, 3
HID = D * 4
KVD = NKV * DK
SCALING = 2.0
GQ = NH // NKV
HALF = DK // 2
SB = 256
NSB = S // SB
BLK = 256
NB = (S * TOPK) // BLK + E
P = NB * BLK
NEG = float(np.finfo(np.float32).min)
F32 = jnp.float32
BF16 = jnp.bfloat16
HI = jax.lax.Precision.HIGHEST

NC, NS, L = 2, 16, 16
NW = NC * NS
TPW = S // NW
HROWS = 32


def _merge_body(wq_ref, wk_ref, wv_ref, gw_ref, qa_ref, qb_ref, ka_ref,
                kb_ref, va_ref, vb_ref, ga_ref, gb_ref, oq, ok, ov, og):
    def m(w0t, a, bm, out, dt):
        up = jnp.dot(a[...], bm[...], preferred_element_type=F32)
        out[...] = (w0t[...] + SCALING * up).astype(dt)
    m(wq_ref, qa_ref, qb_ref, oq, BF16)
    m(wk_ref, ka_ref, kb_ref, ok, BF16)
    m(wv_ref, va_ref, vb_ref, ov, BF16)
    m(gw_ref, ga_ref, gb_ref, og, F32)


def _merge(wqT, wkT, wvT, gwT, qA, qB, kA, kB, vA, vB, gA, gB):
    return pl.pallas_call(
        _merge_body,
        out_shape=(jax.ShapeDtypeStruct((D, D), BF16),
                   jax.ShapeDtypeStruct((D, KVD), BF16),
                   jax.ShapeDtypeStruct((D, KVD), BF16),
                   jax.ShapeDtypeStruct((D, E), F32)),
    )(wqT, wkT, wvT, gwT, qA, qB, kA, kB, vA, vB, gA, gB)


def _qkv_body(x_ref, cos_ref, sin_ref, n1_ref, wq_ref, wk_ref, wv_ref,
              q_out, k_out, v_out):
    xb = x_ref[...]
    ms = jnp.mean(xb * xb, axis=-1, keepdims=True)
    h = (xb * lax.rsqrt(ms + 1e-6) * n1_ref[...]).astype(BF16)
    q = jnp.dot(h, wq_ref[...], preferred_element_type=F32)
    k = jnp.dot(h, wk_ref[...], preferred_element_type=F32)
    v = jnp.dot(h, wv_ref[...], preferred_element_type=F32)
    cos = cos_ref[...]
    sin = sin_ref[...]
    c1, c2 = cos[:, :HALF], cos[:, HALF:]
    s1, s2 = sin[:, :HALF], sin[:, HALF:]
    for hh in range(NH):
        qh = q[:, hh * DK:(hh + 1) * DK]
        q1, q2 = qh[:, :HALF], qh[:, HALF:]
        q_out[hh] = jnp.concatenate(
            [q1 * c1 - q2 * s1, q2 * c2 + q1 * s2], axis=-1).astype(BF16)
    for hh in range(NKV):
        kh = k[:, hh * DK:(hh + 1) * DK]
        k1, k2 = kh[:, :HALF], kh[:, HALF:]
        k_out[hh] = jnp.concatenate(
            [k1 * c1 - k2 * s1, k2 * c2 + k1 * s2], axis=-1).astype(BF16)
        v_out[hh] = v[:, hh * DK:(hh + 1) * DK].astype(BF16)


def _qkv(x2d, cos, sin, n1, wqT, wkT, wvT):
    return pl.pallas_call(
        _qkv_body,
        grid=(NSB,),
        in_specs=[
            pl.BlockSpec((SB, D), lambda i: (i, 0)),
            pl.BlockSpec((SB, DK), lambda i: (i, 0)),
            pl.BlockSpec((SB, DK), lambda i: (i, 0)),
            pl.BlockSpec((1, D), lambda i: (0, 0)),
            pl.BlockSpec((D, D), lambda i: (0, 0)),
            pl.BlockSpec((D, KVD), lambda i: (0, 0)),
            pl.BlockSpec((D, KVD), lambda i: (0, 0)),
        ],
        out_specs=[
            pl.BlockSpec((NH, SB, DK), lambda i: (0, i, 0)),
            pl.BlockSpec((NKV, SB, DK), lambda i: (0, i, 0)),
            pl.BlockSpec((NKV, SB, DK), lambda i: (0, i, 0)),
        ],
        out_shape=(jax.ShapeDtypeStruct((NH, S, DK), BF16),
                   jax.ShapeDtypeStruct((NKV, S, DK), BF16),
                   jax.ShapeDtypeStruct((NKV, S, DK), BF16)),
    )(x2d, cos, sin, n1, wqT, wkT, wvT)


def _attn_body(q_ref, k_ref, v_ref, o_ref):
    iq = pl.program_id(1)
    q = q_ref[0]
    k = k_ref[0]
    s = lax.dot_general(q, k, (((1,), (1,)), ((), ())),
                        preferred_element_type=F32)
    s = s * (1.0 / np.sqrt(DK))
    row = iq * SB + lax.broadcasted_iota(jnp.int32, (SB, S), 0)
    col = lax.broadcasted_iota(jnp.int32, (SB, S), 1)
    s = jnp.where(col <= row, s, NEG)
    m = jnp.max(s, axis=-1, keepdims=True)
    p = jnp.exp(s - m)
    lsum = jnp.sum(p, axis=-1, keepdims=True)
    attn = (p / lsum).astype(BF16)
    o_ref[0] = jnp.dot(attn, v_ref[0],
                       preferred_element_type=F32).astype(BF16)


def _attn(qr, kr, vr):
    return pl.pallas_call(
        _attn_body,
        grid=(NH, NSB),
        in_specs=[
            pl.BlockSpec((1, SB, DK), lambda h, i: (h, i, 0)),
            pl.BlockSpec((1, S, DK), lambda h, i: (h // GQ, 0, 0)),
            pl.BlockSpec((1, S, DK), lambda h, i: (h // GQ, 0, 0)),
        ],
        out_specs=pl.BlockSpec((1, SB, DK), lambda h, i: (h, i, 0)),
        out_shape=jax.ShapeDtypeStruct((NH, S, DK), BF16),
    )(qr, kr, vr)


def _post_body(ctx_ref, x_ref, wo_ref, n2_ref, temb_ref, tid_ref, gw_ref,
               x2_out, xf_out, i1_out, i2_out, w0_out, w1_out):
    x2 = x_ref[...] + jnp.dot(ctx_ref[...], wo_ref[...],
                              preferred_element_type=F32)
    x2_out[...] = x2
    ms = jnp.mean(x2 * x2, axis=-1, keepdims=True)
    h2 = x2 * lax.rsqrt(ms + 1e-6) * n2_ref[...]
    tid = tid_ref[...]
    toh = (tid == lax.broadcasted_iota(jnp.int32, (SB, E), 1)).astype(F32)
    xf = h2 + lax.dot_general(toh, temb_ref[...], (((1,), (0,)), ((), ())),
                              precision=HI, preferred_element_type=F32)
    xf_out[...] = xf
    gl = jnp.dot(xf, gw_ref[...], preferred_element_type=F32)
    e_iota = lax.broadcasted_iota(jnp.int32, (SB, E), 1)
    m1 = jnp.max(gl, axis=-1, keepdims=True)
    i1 = jnp.min(jnp.where(gl == m1, e_iota, E), axis=-1, keepdims=True)
    glm = jnp.where(e_iota == i1, NEG, gl)
    m2 = jnp.max(glm, axis=-1, keepdims=True)
    i2 = jnp.min(jnp.where(glm == m2, e_iota, E), axis=-1, keepdims=True)
    ev = jnp.exp(m2 - m1)
    w0_out[...] = 1.0 / (1.0 + ev)
    w1_out[...] = ev / (1.0 + ev)
    i1_out[...] = i1.astype(jnp.int32)
    i2_out[...] = i2.astype(jnp.int32)


def _post(ctx2d, x2d, woT, n2, temb_p, tid, gwT):
    return pl.pallas_call(
        _post_body,
        grid=(NSB,),
        in_specs=[
            pl.BlockSpec((SB, D), lambda i: (i, 0)),
            pl.BlockSpec((SB, D), lambda i: (i, 0)),
            pl.BlockSpec((D, D), lambda i: (0, 0)),
            pl.BlockSpec((1, D), lambda i: (0, 0)),
            pl.BlockSpec((E, D), lambda i: (0, 0)),
            pl.BlockSpec((SB, 1), lambda i: (i, 0)),
            pl.BlockSpec((D, E), lambda i: (0, 0)),
        ],
        out_specs=[
            pl.BlockSpec((SB, D), lambda i: (i, 0)),
            pl.BlockSpec((SB, D), lambda i: (i, 0)),
            pl.BlockSpec((SB, 1), lambda i: (i, 0)),
            pl.BlockSpec((SB, 1), lambda i: (i, 0)),
            pl.BlockSpec((SB, 1), lambda i: (i, 0)),
            pl.BlockSpec((SB, 1), lambda i: (i, 0)),
        ],
        out_shape=(jax.ShapeDtypeStruct((S, D), F32),
                   jax.ShapeDtypeStruct((S, D), F32),
                   jax.ShapeDtypeStruct((S, 1), jnp.int32),
                   jax.ShapeDtypeStruct((S, 1), jnp.int32),
                   jax.ShapeDtypeStruct((S, 1), F32),
                   jax.ShapeDtypeStruct((S, 1), F32)),
    )(ctx2d, x2d, woT, n2, temb_p, tid, gwT)


def _rank_body(i1_ref, i2_ref, r1_out, r2_out, st_out, be_out, nb_out, base):
    step = pl.program_id(0)

    @pl.when(step == 0)
    def _():
        base[...] = jnp.zeros((1, E), F32)

    p = jnp.concatenate([i1_ref[...], i2_ref[...]], axis=0)
    oh = (p == lax.broadcasted_iota(jnp.int32, (2 * SB, E), 1)).astype(F32)
    ri = lax.broadcasted_iota(jnp.int32, (2 * SB, 2 * SB), 0)
    ci = lax.broadcasted_iota(jnp.int32, (2 * SB, 2 * SB), 1)
    lstrict = (ci < ri).astype(F32)
    prior = lax.dot_general(lstrict, oh, (((1,), (0,)), ((), ())),
                            precision=HI, preferred_element_type=F32)
    rank_all = prior + base[...]
    r = jnp.sum(rank_all * oh, axis=-1, keepdims=True).astype(jnp.int32)
    r1_out[...] = r[:SB]
    r2_out[...] = r[SB:]
    base[...] = base[...] + jnp.sum(oh, axis=0, keepdims=True)

    @pl.when(step == NSB - 1)
    def _():
        cntf = base[...]
        nbe = jnp.floor((cntf + (BLK - 1)) / BLK)
        li = lax.broadcasted_iota(jnp.int32, (E, E), 0)
        lj = lax.broadcasted_iota(jnp.int32, (E, E), 1)
        mstrict = (li < lj).astype(F32)
        sbk = lax.dot_general(nbe, mstrict, (((1,), (0,)), ((), ())),
                              precision=HI, preferred_element_type=F32)
        st_out[...] = jnp.concatenate(
            [sbk * BLK, jnp.zeros((1, E), F32)], axis=1).astype(jnp.int32)
        nbt = jnp.sum(nbe, axis=-1, keepdims=True)
        nb_out[...] = nbt.astype(jnp.int32)
        bio = lax.broadcasted_iota(jnp.int32, (NB, E), 0).astype(F32)
        bcl = jnp.minimum(bio, jnp.broadcast_to(nbt, (NB, E)) - 1.0)
        sbk_b = jnp.broadcast_to(sbk, (NB, E))
        be = jnp.sum((sbk_b <= bcl).astype(F32), axis=-1, keepdims=True) - 1.0
        be_out[...] = be.astype(jnp.int32)


def _rank(i1, i2):
    return pl.pallas_call(
        _rank_body,
        grid=(NSB,),
        in_specs=[
            pl.BlockSpec((SB, 1), lambda i: (i, 0)),
            pl.BlockSpec((SB, 1), lambda i: (i, 0)),
        ],
        out_specs=[
            pl.BlockSpec((SB, 1), lambda i: (i, 0)),
            pl.BlockSpec((SB, 1), lambda i: (i, 0)),
            pl.BlockSpec((1, 16), lambda i: (0, 0)),
            pl.BlockSpec((NB, 1), lambda i: (0, 0)),
            pl.BlockSpec((1, 1), lambda i: (0, 0)),
        ],
        out_shape=(jax.ShapeDtypeStruct((S, 1), jnp.int32),
                   jax.ShapeDtypeStruct((S, 1), jnp.int32),
                   jax.ShapeDtypeStruct((1, 16), jnp.int32),
                   jax.ShapeDtypeStruct((NB, 1), jnp.int32),
                   jax.ShapeDtypeStruct((1, 1), jnp.int32)),
        scratch_shapes=[pltpu.VMEM((1, E), F32)],
    )(i1, i2)


def _posw_body(i1_ref, i2_ref, r1_ref, r2_ref, w0_ref, w1_ref, st_ref,
               p1_out, p2_out, wr1_out, wr2_out):
    st8 = st_ref[...][:, :E].astype(F32)
    def pos(i_ref, r_ref, out):
        oh = (i_ref[...] == lax.broadcasted_iota(jnp.int32, (SB, E), 1)
              ).astype(F32)
        s = lax.dot_general(oh, st8, (((1,), (1,)), ((), ())),
                            precision=HI, preferred_element_type=F32)
        out[...] = s.astype(jnp.int32) + r_ref[...]
    pos(i1_ref, r1_ref, p1_out)
    pos(i2_ref, r2_ref, p2_out)
    wr1_out[...] = jnp.broadcast_to(
        w0_ref[...].astype(BF16).astype(F32), (SB, L))
    wr2_out[...] = jnp.broadcast_to(
        w1_ref[...].astype(BF16).astype(F32), (SB, L))


def _posw(i1, i2, r1, r2, w0, w1, st16):
    return pl.pallas_call(
        _posw_body,
        grid=(NSB,),
        in_specs=[
            pl.BlockSpec((SB, 1), lambda i: (i, 0)),
            pl.BlockSpec((SB, 1), lambda i: (i, 0)),
            pl.BlockSpec((SB, 1), lambda i: (i, 0)),
            pl.BlockSpec((SB, 1), lambda i: (i, 0)),
            pl.BlockSpec((SB, 1), lambda i: (i, 0)),
            pl.BlockSpec((SB, 1), lambda i: (i, 0)),
            pl.BlockSpec((1, 16), lambda i: (0, 0)),
        ],
        out_specs=[
            pl.BlockSpec((SB, 1), lambda i: (i, 0)),
            pl.BlockSpec((SB, 1), lambda i: (i, 0)),
            pl.BlockSpec((SB, L), lambda i: (i, 0)),
            pl.BlockSpec((SB, L), lambda i: (i, 0)),
        ],
        out_shape=(jax.ShapeDtypeStruct((S, 1), jnp.int32),
                   jax.ShapeDtypeStruct((S, 1), jnp.int32),
                   jax.ShapeDtypeStruct((S, L), F32),
                   jax.ShapeDtypeStruct((S, L), F32)),
    )(i1, i2, r1, r2, w0, w1, st16)


@functools.lru_cache(maxsize=None)
def _get_dispatch_sc():
    mesh = plsc.VectorSubcoreMesh(core_axis_name="c", subcore_axis_name="s")

    @functools.partial(
        pl.kernel,
        out_type=jax.ShapeDtypeStruct((P, D), F32),
        mesh=mesh,
        scratch_types=[pltpu.VMEM((TPW, D), F32),
                       pltpu.VMEM((TPW,), jnp.int32),
                       pltpu.VMEM((TPW,), jnp.int32),
                       pltpu.SemaphoreType.DMA],
    )
    def _dispatch_sc(xf_hbm, p1_hbm, p2_hbm, xs_hbm, xrows, iv1, iv2, sem):
        wid = lax.axis_index("s") * NC + lax.axis_index("c")
        base = wid * TPW
        pltpu.sync_copy(xf_hbm.at[pl.ds(base, TPW)], xrows)
        pltpu.sync_copy(p1_hbm.at[pl.ds(base, TPW)], iv1)
        pltpu.sync_copy(p2_hbm.at[pl.ds(base, TPW)], iv2)
        pltpu.async_copy(xrows, xs_hbm.at[iv1], sem).wait()
        pltpu.async_copy(xrows, xs_hbm.at[iv2], sem).wait()

    return _dispatch_sc


def _ffn1_body(be_ref, nb_ref, xs_ref, w1_ref, b1_ref, h1_out):
    b = pl.program_id(0)

    @pl.when(b < nb_ref[0])
    def _():
        xb = xs_ref[...].astype(BF16)
        h1 = lax.dot_general(xb, w1_ref[0], (((1,), (1,)), ((), ())),
                             preferred_element_type=F32)
        h1_out[...] = (h1 + b1_ref[0]).astype(BF16)


def _ffn1(be, nb, xs, w1b, b1):
    return pl.pallas_call(
        _ffn1_body,
        grid_spec=pltpu.PrefetchScalarGridSpec(
            num_scalar_prefetch=2,
            grid=(NB,),
            in_specs=[
                pl.BlockSpec((BLK, D), lambda b, be, nb: (b, 0)),
                pl.BlockSpec((1, HID, D), lambda b, be, nb: (be[b], 0, 0)),
                pl.BlockSpec((1, 1, HID), lambda b, be, nb: (be[b], 0, 0)),
            ],
            out_specs=pl.BlockSpec((BLK, HID), lambda b, be, nb: (b, 0)),
        ),
        out_shape=jax.ShapeDtypeStruct((P, HID), BF16),
    )(be, nb, xs, w1b, b1)


def _ffn2_body(be_ref, nb_ref, h1_ref, wg_ref, wv_ref, bg_ref, bv_ref,
               ys_out):
    b = pl.program_id(0)

    @pl.when(b < nb_ref[0])
    def _():
        h1 = h1_ref[...]
        go = lax.dot_general(h1, wg_ref[0], (((1,), (1,)), ((), ())),
                             preferred_element_type=F32) + bg_ref[0]
        vo = lax.dot_general(h1, wv_ref[0], (((1,), (1,)), ((), ())),
                             preferred_element_type=F32) + bv_ref[0]
        eo = go * (1.0 / (1.0 + jnp.exp(-go))) * vo
        ys_out[...] = eo.astype(BF16).astype(F32)


def _ffn2(be, nb, h1, wgb, wvb, bg, bv):
    return pl.pallas_call(
        _ffn2_body,
        grid_spec=pltpu.PrefetchScalarGridSpec(
            num_scalar_prefetch=2,
            grid=(NB,),
            in_specs=[
                pl.BlockSpec((BLK, HID), lambda b, be, nb: (b, 0)),
                pl.BlockSpec((1, D, HID), lambda b, be, nb: (be[b], 0, 0)),
                pl.BlockSpec((1, D, HID), lambda b, be, nb: (be[b], 0, 0)),
                pl.BlockSpec((1, 1, D), lambda b, be, nb: (be[b], 0, 0)),
                pl.BlockSpec((1, 1, D), lambda b, be, nb: (be[b], 0, 0)),
            ],
            out_specs=pl.BlockSpec((BLK, D), lambda b, be, nb: (b, 0)),
        ),
        out_shape=jax.ShapeDtypeStruct((P, D), F32),
    )(be, nb, h1, wgb, wvb, bg, bv)


@functools.lru_cache(maxsize=None)
def _get_combine_sc():
    mesh = plsc.VectorSubcoreMesh(core_axis_name="c", subcore_axis_name="s")

    @functools.partial(
        pl.kernel,
        out_type=jax.ShapeDtypeStruct((S, D), F32),
        mesh=mesh,
        scratch_types=[pltpu.VMEM((HROWS, D), F32),
                       pltpu.VMEM((HROWS, D), F32),
                       pltpu.VMEM((HROWS, D), F32),
                       pltpu.VMEM((HROWS,), jnp.int32),
                       pltpu.VMEM((HROWS,), jnp.int32),
                       pltpu.VMEM((HROWS, L), F32),
                       pltpu.VMEM((HROWS, L), F32),
                       pltpu.SemaphoreType.DMA],
    )
    def _combine_sc(x2_hbm, ys_hbm, pos1_hbm, pos2_hbm, wr1_hbm, wr2_hbm,
                    out_hbm, xr, y1, y2, p1, p2, wv1, wv2, sem):
        wid = lax.axis_index("s") * NC + lax.axis_index("c")
        for half in range(TPW // HROWS):
            base = wid * TPW + half * HROWS
            pltpu.sync_copy(x2_hbm.at[pl.ds(base, HROWS)], xr)
            pltpu.sync_copy(pos1_hbm.at[pl.ds(base, HROWS)], p1)
            pltpu.sync_copy(pos2_hbm.at[pl.ds(base, HROWS)], p2)
            pltpu.sync_copy(wr1_hbm.at[pl.ds(base, HROWS)], wv1)
            pltpu.sync_copy(wr2_hbm.at[pl.ds(base, HROWS)], wv2)
            pltpu.async_copy(ys_hbm.at[p1], y1, sem).wait()
            pltpu.async_copy(ys_hbm.at[p2], y2, sem).wait()
            for j in range(HROWS):
                xrj = xr.at[j]
                y1j = y1.at[j]
                y2j = y2.at[j]
                w0v = wv1[j]
                w1v = wv2[j]

                def body(c, carry, xrj=xrj, y1j=y1j, y2j=y2j,
                         w0v=w0v, w1v=w1v):
                    sl = pl.ds(c * L, L)
                    xrj[sl] = xrj[sl] + w0v * y1j[sl] + w1v * y2j[sl]
                    return carry

                lax.fori_loop(0, D // L, body, 0)
            pltpu.sync_copy(xr, out_hbm.at[pl.ds(base, HROWS)])

    return _combine_sc


def kernel(x, freqs_cos, freqs_sin, task_ids, n1w, n2w, Wq, Wk, Wv, Wo,
           qA, qB, kA, kB, vA, vB, gateW, gA, gB, temb, W1, b1, Wg, bg,
           Wve, bv):
    x2d = x.reshape(S, D)
    cos = freqs_cos.reshape(S, DK)
    sin = freqs_sin.reshape(S, DK)
    tid = task_ids.reshape(S, 1).astype(jnp.int32)

    wqT, wkT, wvT, gwT = _merge(Wq.T, Wk.T, Wv.T, gateW.T,
                                qA, qB, kA, kB, vA, vB, gA, gB)
    qr, kr, vr = _qkv(x2d, cos, sin, n1w.reshape(1, D), wqT, wkT, wvT)
    ctx = _attn(qr, kr, vr)
    ctx2d = ctx.transpose(1, 0, 2).reshape(S, D)
    temb_p = jnp.pad(temb, ((0, E - NT), (0, 0)))
    x2, xf, i1, i2, w0, w1 = _post(ctx2d, x2d, Wo.T.astype(BF16),
                                   n2w.reshape(1, D), temb_p, tid, gwT)
    r1, r2, st16, be, nb = _rank(i1, i2)
    pos1, pos2, wr1, wr2 = _posw(i1, i2, r1, r2, w0, w1, st16)
    xs = _get_dispatch_sc()(xf, pos1.reshape(S), pos2.reshape(S))
    h1 = _ffn1(be.reshape(NB), nb.reshape(1), xs, W1.astype(BF16),
               b1.reshape(E, 1, HID))
    ys = _ffn2(be.reshape(NB), nb.reshape(1), h1, Wg.astype(BF16),
               Wve.astype(BF16), bg.reshape(E, 1, D), bv.reshape(E, 1, D))
    out = _get_combine_sc()(x2, ys, pos1.reshape(S), pos2.reshape(S),
                            wr1, wr2)
    return out.reshape(1, S, D)


# speedup vs baseline: 1.4905x; 1.1148x over previous
import functools

import jax
import jax.numpy as jnp
import numpy as np
from jax import lax
from jax.experimental import pallas as pl
from jax.experimental.pallas import tpu as pltpu
from jax.experimental.pallas import tpu_sc as plsc

S, D = 2048, 1024
NH, NKV, DK = 16, 4, 64
E, TOPK, R, NT = 8, 2, 16, 3
HID = D * 4
KVD = NKV * DK
SCALING = 2.0
GQ = NH // NKV
GD = GQ * DK
HALF = DK // 2
SB = 256
NSB = S // SB
BLK = 256
NB = (S * TOPK) // BLK + E
P = NB * BLK
NEG = float(np.finfo(np.float32).min)
F32 = jnp.float32
BF16 = jnp.bfloat16
HI = jax.lax.Precision.HIGHEST

NC, NS, L = 2, 16, 16
NW = NC * NS
TPW = S // NW
HROWS = 32


def _merge_body(wq_ref, wk_ref, wv_ref, gw_ref, wo_ref, qa_ref, qb_ref,
                ka_ref, kb_ref, va_ref, vb_ref, ga_ref, gb_ref,
                oq, ok, ov, og, oo):
    def m(w0, a, bm, out, dt):
        up = lax.dot_general(bm[...], a[...], (((0,), (1,)), ((), ())),
                             preferred_element_type=F32)
        out[...] = (w0[...] + SCALING * up).astype(dt)
    m(wq_ref, qa_ref, qb_ref, oq, BF16)
    m(wk_ref, ka_ref, kb_ref, ok, BF16)
    m(wv_ref, va_ref, vb_ref, ov, BF16)
    m(gw_ref, ga_ref, gb_ref, og, F32)
    oo[...] = wo_ref[...].astype(BF16)


def _merge(Wq, Wk, Wv, gateW, Wo, qA, qB, kA, kB, vA, vB, gA, gB):
    return pl.pallas_call(
        _merge_body,
        out_shape=(jax.ShapeDtypeStruct((D, D), BF16),
                   jax.ShapeDtypeStruct((KVD, D), BF16),
                   jax.ShapeDtypeStruct((KVD, D), BF16),
                   jax.ShapeDtypeStruct((E, D), F32),
                   jax.ShapeDtypeStruct((D, D), BF16)),
    )(Wq, Wk, Wv, gateW, Wo, qA, qB, kA, kB, vA, vB, gA, gB)


def _qkv_body(x_ref, cos_ref, sin_ref, n1_ref, wq_ref, wk_ref, wv_ref,
              q_out, k_out, v_out):
    xb = x_ref[...]
    ms = jnp.mean(xb * xb, axis=-1, keepdims=True)
    h = (xb * lax.rsqrt(ms + 1e-6) * n1_ref[...]).astype(BF16)
    q = lax.dot_general(h, wq_ref[...], (((1,), (1,)), ((), ())),
                        preferred_element_type=F32)
    k = lax.dot_general(h, wk_ref[...], (((1,), (1,)), ((), ())),
                        preferred_element_type=F32)
    v = lax.dot_general(h, wv_ref[...], (((1,), (1,)), ((), ())),
                        preferred_element_type=F32)
    cos = cos_ref[...]
    sin = sin_ref[...]
    c1, c2 = cos[:, :HALF], cos[:, HALF:]
    s1, s2 = sin[:, :HALF], sin[:, HALF:]
    pieces = []
    for hh in range(NH):
        qh = q[:, hh * DK:(hh + 1) * DK]
        q1, q2 = qh[:, :HALF], qh[:, HALF:]
        pieces.append(jnp.concatenate(
            [q1 * c1 - q2 * s1, q2 * c2 + q1 * s2], axis=-1))
    q_out[...] = jnp.concatenate(pieces, axis=-1).astype(BF16)
    for hh in range(NKV):
        kh = k[:, hh * DK:(hh + 1) * DK]
        k1, k2 = kh[:, :HALF], kh[:, HALF:]
        k_out[hh] = jnp.concatenate(
            [k1 * c1 - k2 * s1, k2 * c2 + k1 * s2], axis=-1).astype(BF16)
        v_out[hh] = v[:, hh * DK:(hh + 1) * DK].astype(BF16)


def _qkv(x2d, cos, sin, n1, wq, wk, wv):
    return pl.pallas_call(
        _qkv_body,
        grid=(NSB,),
        in_specs=[
            pl.BlockSpec((SB, D), lambda i: (i, 0)),
            pl.BlockSpec((SB, DK), lambda i: (i, 0)),
            pl.BlockSpec((SB, DK), lambda i: (i, 0)),
            pl.BlockSpec((1, D), lambda i: (0, 0)),
            pl.BlockSpec((D, D), lambda i: (0, 0)),
            pl.BlockSpec((KVD, D), lambda i: (0, 0)),
            pl.BlockSpec((KVD, D), lambda i: (0, 0)),
        ],
        out_specs=[
            pl.BlockSpec((SB, D), lambda i: (i, 0)),
            pl.BlockSpec((NKV, SB, DK), lambda i: (0, i, 0)),
            pl.BlockSpec((NKV, SB, DK), lambda i: (0, i, 0)),
        ],
        out_shape=(jax.ShapeDtypeStruct((S, D), BF16),
                   jax.ShapeDtypeStruct((NKV, S, DK), BF16),
                   jax.ShapeDtypeStruct((NKV, S, DK), BF16)),
    )(x2d, cos, sin, n1, wq, wk, wv)


def _attn_body(q_ref, k_ref, v_ref, o_ref, *, q0, kw):
    i = pl.program_id(1)
    row = (q0 + i) * SB + lax.broadcasted_iota(jnp.int32, (SB, kw), 0)
    col = lax.broadcasted_iota(jnp.int32, (SB, kw), 1)
    causal = col <= row
    qg = q_ref[...] * BF16(0.125)
    k = k_ref[0]
    v = v_ref[0]
    pieces = []
    for hh in range(GQ):
        q = qg[:, hh * DK:(hh + 1) * DK]
        s = lax.dot_general(q, k, (((1,), (1,)), ((), ())),
                            preferred_element_type=F32)
        s = jnp.where(causal, s, NEG)
        m = jnp.max(s, axis=-1, keepdims=True)
        p = jnp.exp(s - m)
        lsum = jnp.sum(p, axis=-1, keepdims=True)
        attn = (p * (1.0 / lsum)).astype(BF16)
        pieces.append(jnp.dot(attn, v, preferred_element_type=F32))
    o_ref[...] = jnp.concatenate(pieces, axis=-1).astype(BF16)


def _attn_part(q2d, kr, vr, q0, nqb, kw):
    body = functools.partial(_attn_body, q0=q0, kw=kw)
    return pl.pallas_call(
        body,
        grid=(NKV, nqb),
        in_specs=[
            pl.BlockSpec((SB, GD), lambda g, i: (q0 + i, g)),
            pl.BlockSpec((1, kw, DK), lambda g, i: (g, 0, 0)),
            pl.BlockSpec((1, kw, DK), lambda g, i: (g, 0, 0)),
        ],
        out_specs=pl.BlockSpec((SB, GD), lambda g, i: (i, g)),
        out_shape=jax.ShapeDtypeStruct((nqb * SB, D), BF16),
    )(q2d, kr, vr)


def _post_body(ctx_ref, x_ref, wo_ref, n2_ref, temb_ref, tid_ref, gw_ref,
               x2_out, xf_out, i1_out, i2_out, w0_out, w1_out,
               r1_out, r2_out, st_out, be_out, nb_out, base):
    step = pl.program_id(0)
    x2 = x_ref[...] + lax.dot_general(ctx_ref[...], wo_ref[...],
                                      (((1,), (1,)), ((), ())),
                                      preferred_element_type=F32)
    x2_out[...] = x2
    ms = jnp.mean(x2 * x2, axis=-1, keepdims=True)
    h2 = x2 * lax.rsqrt(ms + 1e-6) * n2_ref[...]
    tid = tid_ref[...]
    toh = (tid == lax.broadcasted_iota(jnp.int32, (SB, E), 1)).astype(F32)
    xf = h2 + lax.dot_general(toh, temb_ref[...], (((1,), (0,)), ((), ())),
                              precision=HI, preferred_element_type=F32)
    xf_out[...] = xf
    gl = lax.dot_general(xf, gw_ref[...], (((1,), (1,)), ((), ())),
                         preferred_element_type=F32)
    e_iota = lax.broadcasted_iota(jnp.int32, (SB, E), 1)
    m1 = jnp.max(gl, axis=-1, keepdims=True)
    i1 = jnp.min(jnp.where(gl == m1, e_iota, E), axis=-1, keepdims=True)
    glm = jnp.where(e_iota == i1, NEG, gl)
    m2 = jnp.max(glm, axis=-1, keepdims=True)
    i2 = jnp.min(jnp.where(glm == m2, e_iota, E), axis=-1, keepdims=True)
    ev = jnp.exp(m2 - m1)
    w0_out[...] = 1.0 / (1.0 + ev)
    w1_out[...] = ev / (1.0 + ev)
    i1_out[...] = i1.astype(jnp.int32)
    i2_out[...] = i2.astype(jnp.int32)

    @pl.when(step == 0)
    def _():
        base[...] = jnp.zeros((1, E), F32)

    pair = jnp.concatenate([i1, i2], axis=0).astype(jnp.int32)
    oh = (pair == lax.broadcasted_iota(jnp.int32, (2 * SB, E), 1)).astype(F32)
    ri = lax.broadcasted_iota(jnp.int32, (2 * SB, 2 * SB), 0)
    ci = lax.broadcasted_iota(jnp.int32, (2 * SB, 2 * SB), 1)
    lstrict = (ci < ri).astype(F32)
    prior = lax.dot_general(lstrict, oh, (((1,), (0,)), ((), ())),
                            precision=HI, preferred_element_type=F32)
    rank_all = prior + base[...]
    r = jnp.sum(rank_all * oh, axis=-1, keepdims=True).astype(jnp.int32)
    r1_out[...] = r[:SB]
    r2_out[...] = r[SB:]
    base[...] = base[...] + jnp.sum(oh, axis=0, keepdims=True)

    @pl.when(step == NSB - 1)
    def _():
        cntf = base[...]
        nbe = jnp.floor((cntf + (BLK - 1)) / BLK)
        li = lax.broadcasted_iota(jnp.int32, (E, E), 0)
        lj = lax.broadcasted_iota(jnp.int32, (E, E), 1)
        mstrict = (li < lj).astype(F32)
        sbk = lax.dot_general(nbe, mstrict, (((1,), (0,)), ((), ())),
                              precision=HI, preferred_element_type=F32)
        st_out[...] = jnp.concatenate(
            [sbk * BLK, jnp.zeros((1, E), F32)], axis=1).astype(jnp.int32)
        nbt = jnp.sum(nbe, axis=-1, keepdims=True)
        nb_out[...] = nbt.astype(jnp.int32)
        bio = lax.broadcasted_iota(jnp.int32, (NB, E), 0).astype(F32)
        bcl = jnp.minimum(bio, jnp.broadcast_to(nbt, (NB, E)) - 1.0)
        sbk_b = jnp.broadcast_to(sbk, (NB, E))
        be = jnp.sum((sbk_b <= bcl).astype(F32), axis=-1, keepdims=True) - 1.0
        be_out[...] = be.astype(jnp.int32)


def _post(ctx2d, x2d, wo, n2, temb_p, tid, gw):
    return pl.pallas_call(
        _post_body,
        grid=(NSB,),
        in_specs=[
            pl.BlockSpec((SB, D), lambda i: (i, 0)),
            pl.BlockSpec((SB, D), lambda i: (i, 0)),
            pl.BlockSpec((D, D), lambda i: (0, 0)),
            pl.BlockSpec((1, D), lambda i: (0, 0)),
            pl.BlockSpec((E, D), lambda i: (0, 0)),
            pl.BlockSpec((SB, 1), lambda i: (i, 0)),
            pl.BlockSpec((E, D), lambda i: (0, 0)),
        ],
        out_specs=[
            pl.BlockSpec((SB, D), lambda i: (i, 0)),
            pl.BlockSpec((SB, D), lambda i: (i, 0)),
            pl.BlockSpec((SB, 1), lambda i: (i, 0)),
            pl.BlockSpec((SB, 1), lambda i: (i, 0)),
            pl.BlockSpec((SB, 1), lambda i: (i, 0)),
            pl.BlockSpec((SB, 1), lambda i: (i, 0)),
            pl.BlockSpec((SB, 1), lambda i: (i, 0)),
            pl.BlockSpec((SB, 1), lambda i: (i, 0)),
            pl.BlockSpec((1, 16), lambda i: (0, 0)),
            pl.BlockSpec((NB, 1), lambda i: (0, 0)),
            pl.BlockSpec((1, 1), lambda i: (0, 0)),
        ],
        out_shape=(jax.ShapeDtypeStruct((S, D), F32),
                   jax.ShapeDtypeStruct((S, D), F32),
                   jax.ShapeDtypeStruct((S, 1), jnp.int32),
                   jax.ShapeDtypeStruct((S, 1), jnp.int32),
                   jax.ShapeDtypeStruct((S, 1), F32),
                   jax.ShapeDtypeStruct((S, 1), F32),
                   jax.ShapeDtypeStruct((S, 1), jnp.int32),
                   jax.ShapeDtypeStruct((S, 1), jnp.int32),
                   jax.ShapeDtypeStruct((1, 16), jnp.int32),
                   jax.ShapeDtypeStruct((NB, 1), jnp.int32),
                   jax.ShapeDtypeStruct((1, 1), jnp.int32)),
        scratch_shapes=[pltpu.VMEM((1, E), F32)],
    )(ctx2d, x2d, wo, n2, temb_p, tid, gw)


def _posw_body(i1_ref, i2_ref, r1_ref, r2_ref, w0_ref, w1_ref, st_ref,
               p1_out, p2_out, wr1_out, wr2_out):
    st8 = st_ref[...][:, :E].astype(F32)
    def pos(i_ref, r_ref, out):
        oh = (i_ref[...] == lax.broadcasted_iota(jnp.int32, (SB, E), 1)
              ).astype(F32)
        s = lax.dot_general(oh, st8, (((1,), (1,)), ((), ())),
                            precision=HI, preferred_element_type=F32)
        out[...] = s.astype(jnp.int32) + r_ref[...]
    pos(i1_ref, r1_ref, p1_out)
    pos(i2_ref, r2_ref, p2_out)
    wr1_out[...] = jnp.broadcast_to(
        w0_ref[...].astype(BF16).astype(F32), (SB, L))
    wr2_out[...] = jnp.broadcast_to(
        w1_ref[...].astype(BF16).astype(F32), (SB, L))


def _posw(i1, i2, r1, r2, w0, w1, st16):
    return pl.pallas_call(
        _posw_body,
        grid=(NSB,),
        in_specs=[
            pl.BlockSpec((SB, 1), lambda i: (i, 0)),
            pl.BlockSpec((SB, 1), lambda i: (i, 0)),
            pl.BlockSpec((SB, 1), lambda i: (i, 0)),
            pl.BlockSpec((SB, 1), lambda i: (i, 0)),
            pl.BlockSpec((SB, 1), lambda i: (i, 0)),
            pl.BlockSpec((SB, 1), lambda i: (i, 0)),
            pl.BlockSpec((1, 16), lambda i: (0, 0)),
        ],
        out_specs=[
            pl.BlockSpec((SB, 1), lambda i: (i, 0)),
            pl.BlockSpec((SB, 1), lambda i: (i, 0)),
            pl.BlockSpec((SB, L), lambda i: (i, 0)),
            pl.BlockSpec((SB, L), lambda i: (i, 0)),
        ],
        out_shape=(jax.ShapeDtypeStruct((S, 1), jnp.int32),
                   jax.ShapeDtypeStruct((S, 1), jnp.int32),
                   jax.ShapeDtypeStruct((S, L), F32),
                   jax.ShapeDtypeStruct((S, L), F32)),
    )(i1, i2, r1, r2, w0, w1, st16)


@functools.lru_cache(maxsize=None)
def _get_dispatch_sc():
    mesh = plsc.VectorSubcoreMesh(core_axis_name="c", subcore_axis_name="s")

    @functools.partial(
        pl.kernel,
        out_type=jax.ShapeDtypeStruct((P, D), F32),
        mesh=mesh,
        scratch_types=[pltpu.VMEM((TPW, D), F32),
                       pltpu.VMEM((TPW,), jnp.int32),
                       pltpu.VMEM((TPW,), jnp.int32),
                       pltpu.SemaphoreType.DMA],
    )
    def _dispatch_sc(xf_hbm, p1_hbm, p2_hbm, xs_hbm, xrows, iv1, iv2, sem):
        wid = lax.axis_index("s") * NC + lax.axis_index("c")
        base = wid * TPW
        pltpu.sync_copy(xf_hbm.at[pl.ds(base, TPW)], xrows)
        pltpu.sync_copy(p1_hbm.at[pl.ds(base, TPW)], iv1)
        pltpu.sync_copy(p2_hbm.at[pl.ds(base, TPW)], iv2)
        c1 = pltpu.async_copy(xrows, xs_hbm.at[iv1], sem)
        c2 = pltpu.async_copy(xrows, xs_hbm.at[iv2], sem)
        c1.wait()
        c2.wait()

    return _dispatch_sc


def _ffn1_body(be_ref, nb_ref, xs_ref, w1_ref, b1_ref, h1_out):
    b = pl.program_id(0)

    @pl.when(b < nb_ref[0])
    def _():
        xb = xs_ref[...].astype(BF16)
        h1 = lax.dot_general(xb, w1_ref[0], (((1,), (1,)), ((), ())),
                             preferred_element_type=F32)
        h1_out[...] = (h1 + b1_ref[0]).astype(BF16)


def _ffn1(be, nb, xs, w1b, b1):
    return pl.pallas_call(
        _ffn1_body,
        grid_spec=pltpu.PrefetchScalarGridSpec(
            num_scalar_prefetch=2,
            grid=(NB,),
            in_specs=[
                pl.BlockSpec((BLK, D),
                             lambda b, be, nb: (jnp.minimum(b, nb[0] - 1), 0)),
                pl.BlockSpec((1, HID, D), lambda b, be, nb: (be[b], 0, 0)),
                pl.BlockSpec((1, 1, HID), lambda b, be, nb: (be[b], 0, 0)),
            ],
            out_specs=pl.BlockSpec(
                (BLK, HID), lambda b, be, nb: (jnp.minimum(b, nb[0] - 1), 0)),
        ),
        out_shape=jax.ShapeDtypeStruct((P, HID), BF16),
    )(be, nb, xs, w1b, b1)


def _ffn2_body(be_ref, nb_ref, h1_ref, wg_ref, wv_ref, bg_ref, bv_ref,
               ys_out):
    b = pl.program_id(0)

    @pl.when(b < nb_ref[0])
    def _():
        h1 = h1_ref[...]
        go = lax.dot_general(h1, wg_ref[0], (((1,), (1,)), ((), ())),
                             preferred_element_type=F32) + bg_ref[0]
        vo = lax.dot_general(h1, wv_ref[0], (((1,), (1,)), ((), ())),
                             preferred_element_type=F32) + bv_ref[0]
        eo = go * (1.0 / (1.0 + jnp.exp(-go))) * vo
        ys_out[...] = eo.astype(BF16).astype(F32)


def _ffn2(be, nb, h1, wgb, wvb, bg, bv):
    return pl.pallas_call(
        _ffn2_body,
        grid_spec=pltpu.PrefetchScalarGridSpec(
            num_scalar_prefetch=2,
            grid=(NB,),
            in_specs=[
                pl.BlockSpec((BLK, HID),
                             lambda b, be, nb: (jnp.minimum(b, nb[0] - 1), 0)),
                pl.BlockSpec((1, D, HID), lambda b, be, nb: (be[b], 0, 0)),
                pl.BlockSpec((1, D, HID), lambda b, be, nb: (be[b], 0, 0)),
                pl.BlockSpec((1, 1, D), lambda b, be, nb: (be[b], 0, 0)),
                pl.BlockSpec((1, 1, D), lambda b, be, nb: (be[b], 0, 0)),
            ],
            out_specs=pl.BlockSpec(
                (BLK, D), lambda b, be, nb: (jnp.minimum(b, nb[0] - 1), 0)),
        ),
        out_shape=jax.ShapeDtypeStruct((P, D), F32),
    )(be, nb, h1, wgb, wvb, bg, bv)


@functools.lru_cache(maxsize=None)
def _get_combine_sc():
    mesh = plsc.VectorSubcoreMesh(core_axis_name="c", subcore_axis_name="s")

    @functools.partial(
        pl.kernel,
        out_type=jax.ShapeDtypeStruct((S, D), F32),
        mesh=mesh,
        scratch_types=[pltpu.VMEM((HROWS, D), F32),
                       pltpu.VMEM((HROWS, D), F32),
                       pltpu.VMEM((HROWS, D), F32),
                       pltpu.VMEM((HROWS,), jnp.int32),
                       pltpu.VMEM((HROWS,), jnp.int32),
                       pltpu.VMEM((HROWS, L), F32),
                       pltpu.VMEM((HROWS, L), F32),
                       pltpu.SemaphoreType.DMA],
    )
    def _combine_sc(x2_hbm, ys_hbm, pos1_hbm, pos2_hbm, wr1_hbm, wr2_hbm,
                    out_hbm, xr, y1, y2, p1, p2, wv1, wv2, sem):
        wid = lax.axis_index("s") * NC + lax.axis_index("c")
        for half in range(TPW // HROWS):
            base = wid * TPW + half * HROWS
            pltpu.sync_copy(x2_hbm.at[pl.ds(base, HROWS)], xr)
            pltpu.sync_copy(pos1_hbm.at[pl.ds(base, HROWS)], p1)
            pltpu.sync_copy(pos2_hbm.at[pl.ds(base, HROWS)], p2)
            pltpu.sync_copy(wr1_hbm.at[pl.ds(base, HROWS)], wv1)
            pltpu.sync_copy(wr2_hbm.at[pl.ds(base, HROWS)], wv2)
            c1 = pltpu.async_copy(ys_hbm.at[p1], y1, sem)
            c2 = pltpu.async_copy(ys_hbm.at[p2], y2, sem)
            c1.wait()
            c2.wait()
            for j in range(HROWS):
                xrj = xr.at[j]
                y1j = y1.at[j]
                y2j = y2.at[j]
                w0v = wv1[j]
                w1v = wv2[j]

                def body(c, carry, xrj=xrj, y1j=y1j, y2j=y2j,
                         w0v=w0v, w1v=w1v):
                    sl = pl.ds(c * L, L)
                    xrj[sl] = xrj[sl] + w0v * y1j[sl] + w1v * y2j[sl]
                    return carry

                lax.fori_loop(0, D // L, body, 0)
            pltpu.sync_copy(xr, out_hbm.at[pl.ds(base, HROWS)])

    return _combine_sc


def kernel(x, freqs_cos, freqs_sin, task_ids, n1w, n2w, Wq, Wk, Wv, Wo,
           qA, qB, kA, kB, vA, vB, gateW, gA, gB, temb, W1, b1, Wg, bg,
           Wve, bv):
    x2d = x.reshape(S, D)
    cos = freqs_cos.reshape(S, DK)
    sin = freqs_sin.reshape(S, DK)
    tid = task_ids.reshape(S, 1).astype(jnp.int32)

    wq, wk, wv, gw, wo = _merge(Wq, Wk, Wv, gateW, Wo,
                                qA, qB, kA, kB, vA, vB, gA, gB)
    q2d, kr, vr = _qkv(x2d, cos, sin, n1w.reshape(1, D), wq, wk, wv)
    ctx_a = _attn_part(q2d, kr, vr, 0, NSB // 2, S // 2)
    ctx_b = _attn_part(q2d, kr, vr, NSB // 2, NSB // 2, S)
    ctx2d = jnp.concatenate([ctx_a, ctx_b], axis=0)
    temb_p = jnp.pad(temb, ((0, E - NT), (0, 0)))
    (x2, xf, i1, i2, w0, w1, r1, r2, st16, be, nb) = _post(
        ctx2d, x2d, wo, n2w.reshape(1, D), temb_p, tid, gw)
    pos1, pos2, wr1, wr2 = _posw(i1, i2, r1, r2, w0, w1, st16)
    xs = _get_dispatch_sc()(xf, pos1.reshape(S), pos2.reshape(S))
    h1 = _ffn1(be.reshape(NB), nb.reshape(1), xs, W1.astype(BF16),
               b1.reshape(E, 1, HID))
    ys = _ffn2(be.reshape(NB), nb.reshape(1), h1, Wg.astype(BF16),
               Wve.astype(BF16), bg.reshape(E, 1, D), bv.reshape(E, 1, D))
    out = _get_combine_sc()(x2, ys, pos1.reshape(S), pos2.reshape(S),
                            wr1, wr2)
    return out.reshape(1, S, D)


# speedup vs baseline: 1.5902x; 1.0669x over previous
import functools

import jax
import jax.numpy as jnp
import numpy as np
from jax import lax
from jax.experimental import pallas as pl
from jax.experimental.pallas import tpu as pltpu
from jax.experimental.pallas import tpu_sc as plsc

S, D = 2048, 1024
NH, NKV, DK = 16, 4, 64
E, TOPK, R, NT = 8, 2, 16, 3
HID = D * 4
KVD = NKV * DK
SCALING = 2.0
GQ = NH // NKV
GD = GQ * DK
HALF = DK // 2
SB = 256
NSB = S // SB
BLK = 256
NB = (S * TOPK) // BLK + E
P = NB * BLK
NEG = float(np.finfo(np.float32).min)
F32 = jnp.float32
BF16 = jnp.bfloat16
HI = jax.lax.Precision.HIGHEST

NC, NS, L = 2, 16, 16
NW = NC * NS
TPW = S // NW
HROWS = 32


def _merge_body(wq_ref, wk_ref, wv_ref, gw_ref, wo_ref, qa_ref, qb_ref,
                ka_ref, kb_ref, va_ref, vb_ref, ga_ref, gb_ref,
                oq, ok, ov, og, oo):
    def m(w0, a, bm, out, dt):
        up = lax.dot_general(bm[...], a[...], (((0,), (1,)), ((), ())),
                             preferred_element_type=F32)
        out[...] = (w0[...] + SCALING * up).astype(dt)
    m(wq_ref, qa_ref, qb_ref, oq, BF16)
    m(wk_ref, ka_ref, kb_ref, ok, BF16)
    m(wv_ref, va_ref, vb_ref, ov, BF16)
    m(gw_ref, ga_ref, gb_ref, og, F32)
    oo[...] = wo_ref[...].astype(BF16)


def _merge(Wq, Wk, Wv, gateW, Wo, qA, qB, kA, kB, vA, vB, gA, gB):
    return pl.pallas_call(
        _merge_body,
        out_shape=(jax.ShapeDtypeStruct((D, D), BF16),
                   jax.ShapeDtypeStruct((KVD, D), BF16),
                   jax.ShapeDtypeStruct((KVD, D), BF16),
                   jax.ShapeDtypeStruct((E, D), F32),
                   jax.ShapeDtypeStruct((D, D), BF16)),
    )(Wq, Wk, Wv, gateW, Wo, qA, qB, kA, kB, vA, vB, gA, gB)


def _qkv_body(x_ref, cos_ref, sin_ref, n1_ref, wq_ref, wk_ref, wv_ref,
              q_out, k_out, v_out):
    xb = x_ref[...]
    ms = jnp.mean(xb * xb, axis=-1, keepdims=True)
    h = (xb * lax.rsqrt(ms + 1e-6) * n1_ref[...]).astype(BF16)
    q = lax.dot_general(h, wq_ref[...], (((1,), (1,)), ((), ())),
                        preferred_element_type=F32)
    k = lax.dot_general(h, wk_ref[...], (((1,), (1,)), ((), ())),
                        preferred_element_type=F32)
    v = lax.dot_general(h, wv_ref[...], (((1,), (1,)), ((), ())),
                        preferred_element_type=F32)
    cos = cos_ref[...]
    sin = sin_ref[...]
    c1, c2 = cos[:, :HALF], cos[:, HALF:]
    s1, s2 = sin[:, :HALF], sin[:, HALF:]
    pieces = []
    for hh in range(NH):
        qh = q[:, hh * DK:(hh + 1) * DK]
        q1, q2 = qh[:, :HALF], qh[:, HALF:]
        pieces.append(jnp.concatenate(
            [q1 * c1 - q2 * s1, q2 * c2 + q1 * s2], axis=-1))
    q_out[...] = jnp.concatenate(pieces, axis=-1).astype(BF16)
    for hh in range(NKV):
        kh = k[:, hh * DK:(hh + 1) * DK]
        k1, k2 = kh[:, :HALF], kh[:, HALF:]
        k_out[hh] = jnp.concatenate(
            [k1 * c1 - k2 * s1, k2 * c2 + k1 * s2], axis=-1).astype(BF16)
        v_out[hh] = v[:, hh * DK:(hh + 1) * DK].astype(BF16)


def _qkv(x2d, cos, sin, n1, wq, wk, wv):
    return pl.pallas_call(
        _qkv_body,
        grid=(NSB,),
        in_specs=[
            pl.BlockSpec((SB, D), lambda i: (i, 0)),
            pl.BlockSpec((SB, DK), lambda i: (i, 0)),
            pl.BlockSpec((SB, DK), lambda i: (i, 0)),
            pl.BlockSpec((1, D), lambda i: (0, 0)),
            pl.BlockSpec((D, D), lambda i: (0, 0)),
            pl.BlockSpec((KVD, D), lambda i: (0, 0)),
            pl.BlockSpec((KVD, D), lambda i: (0, 0)),
        ],
        out_specs=[
            pl.BlockSpec((SB, D), lambda i: (i, 0)),
            pl.BlockSpec((NKV, SB, DK), lambda i: (0, i, 0)),
            pl.BlockSpec((NKV, SB, DK), lambda i: (0, i, 0)),
        ],
        out_shape=(jax.ShapeDtypeStruct((S, D), BF16),
                   jax.ShapeDtypeStruct((NKV, S, DK), BF16),
                   jax.ShapeDtypeStruct((NKV, S, DK), BF16)),
    )(x2d, cos, sin, n1, wq, wk, wv)


def _attn_body(q_ref, k_ref, v_ref, o_ref, *, q0, kw):
    i = pl.program_id(1)
    row = (q0 + i) * SB + lax.broadcasted_iota(jnp.int32, (SB, kw), 0)
    col = lax.broadcasted_iota(jnp.int32, (SB, kw), 1)
    causal = col <= row
    qg = q_ref[...] * BF16(0.125)
    k = k_ref[0]
    v = v_ref[0]
    pieces = []
    for hh in range(GQ):
        q = qg[:, hh * DK:(hh + 1) * DK]
        s = lax.dot_general(q, k, (((1,), (1,)), ((), ())),
                            preferred_element_type=F32)
        s = jnp.where(causal, s, NEG)
        m = jnp.max(s, axis=-1, keepdims=True)
        p = jnp.exp(s - m)
        lsum = jnp.sum(p, axis=-1, keepdims=True)
        attn = (p * (1.0 / lsum)).astype(BF16)
        pieces.append(jnp.dot(attn, v, preferred_element_type=F32))
    o_ref[...] = jnp.concatenate(pieces, axis=-1).astype(BF16)


def _attn_part(q2d, kr, vr, q0, nqb, kw):
    body = functools.partial(_attn_body, q0=q0, kw=kw)
    return pl.pallas_call(
        body,
        grid=(NKV, nqb),
        in_specs=[
            pl.BlockSpec((SB, GD), lambda g, i: (q0 + i, g)),
            pl.BlockSpec((1, kw, DK), lambda g, i: (g, 0, 0)),
            pl.BlockSpec((1, kw, DK), lambda g, i: (g, 0, 0)),
        ],
        out_specs=pl.BlockSpec((SB, GD), lambda g, i: (i, g)),
        out_shape=jax.ShapeDtypeStruct((nqb * SB, D), BF16),
    )(q2d, kr, vr)


def _post_body(ctx_ref, x_ref, wo_ref, n2_ref, temb_ref, tid_ref, gw_ref,
               x2_out, xf_out, i1_out, i2_out, w0_out, w1_out,
               r1_out, r2_out, st_out, be_out, nb_out, base):
    step = pl.program_id(0)
    x2 = x_ref[...] + lax.dot_general(ctx_ref[...], wo_ref[...],
                                      (((1,), (1,)), ((), ())),
                                      preferred_element_type=F32)
    x2_out[...] = x2
    ms = jnp.mean(x2 * x2, axis=-1, keepdims=True)
    h2 = x2 * lax.rsqrt(ms + 1e-6) * n2_ref[...]
    tid = tid_ref[...]
    toh = (tid == lax.broadcasted_iota(jnp.int32, (SB, E), 1)).astype(F32)
    xf = h2 + lax.dot_general(toh, temb_ref[...], (((1,), (0,)), ((), ())),
                              precision=HI, preferred_element_type=F32)
    xf_out[...] = xf
    gl = lax.dot_general(xf, gw_ref[...], (((1,), (1,)), ((), ())),
                         preferred_element_type=F32)
    e_iota = lax.broadcasted_iota(jnp.int32, (SB, E), 1)
    m1 = jnp.max(gl, axis=-1, keepdims=True)
    i1 = jnp.min(jnp.where(gl == m1, e_iota, E), axis=-1, keepdims=True)
    glm = jnp.where(e_iota == i1, NEG, gl)
    m2 = jnp.max(glm, axis=-1, keepdims=True)
    i2 = jnp.min(jnp.where(glm == m2, e_iota, E), axis=-1, keepdims=True)
    ev = jnp.exp(m2 - m1)
    w0_out[...] = 1.0 / (1.0 + ev)
    w1_out[...] = ev / (1.0 + ev)
    i1_out[...] = i1.astype(jnp.int32)
    i2_out[...] = i2.astype(jnp.int32)

    @pl.when(step == 0)
    def _():
        base[...] = jnp.zeros((1, E), F32)

    pair = jnp.concatenate([i1, i2], axis=0).astype(jnp.int32)
    oh = (pair == lax.broadcasted_iota(jnp.int32, (2 * SB, E), 1)).astype(F32)
    ri = lax.broadcasted_iota(jnp.int32, (2 * SB, 2 * SB), 0)
    ci = lax.broadcasted_iota(jnp.int32, (2 * SB, 2 * SB), 1)
    lstrict = (ci < ri).astype(F32)
    prior = lax.dot_general(lstrict, oh, (((1,), (0,)), ((), ())),
                            precision=HI, preferred_element_type=F32)
    rank_all = prior + base[...]
    r = jnp.sum(rank_all * oh, axis=-1, keepdims=True).astype(jnp.int32)
    r1_out[...] = r[:SB]
    r2_out[...] = r[SB:]
    base[...] = base[...] + jnp.sum(oh, axis=0, keepdims=True)

    @pl.when(step == NSB - 1)
    def _():
        cntf = base[...]
        nbe = jnp.floor((cntf + (BLK - 1)) / BLK)
        li = lax.broadcasted_iota(jnp.int32, (E, E), 0)
        lj = lax.broadcasted_iota(jnp.int32, (E, E), 1)
        mstrict = (li < lj).astype(F32)
        sbk = lax.dot_general(nbe, mstrict, (((1,), (0,)), ((), ())),
                              precision=HI, preferred_element_type=F32)
        st_out[...] = jnp.concatenate(
            [sbk * BLK, jnp.zeros((1, E), F32)], axis=1).astype(jnp.int32)
        nbt = jnp.sum(nbe, axis=-1, keepdims=True)
        nb_out[...] = nbt.astype(jnp.int32)
        bio = lax.broadcasted_iota(jnp.int32, (NB, E), 0).astype(F32)
        bcl = jnp.minimum(bio, jnp.broadcast_to(nbt, (NB, E)) - 1.0)
        sbk_b = jnp.broadcast_to(sbk, (NB, E))
        be = jnp.sum((sbk_b <= bcl).astype(F32), axis=-1, keepdims=True) - 1.0
        be_out[...] = be.astype(jnp.int32)


def _post(ctx2d, x2d, wo, n2, temb_p, tid, gw):
    return pl.pallas_call(
        _post_body,
        grid=(NSB,),
        in_specs=[
            pl.BlockSpec((SB, D), lambda i: (i, 0)),
            pl.BlockSpec((SB, D), lambda i: (i, 0)),
            pl.BlockSpec((D, D), lambda i: (0, 0)),
            pl.BlockSpec((1, D), lambda i: (0, 0)),
            pl.BlockSpec((E, D), lambda i: (0, 0)),
            pl.BlockSpec((SB, 1), lambda i: (i, 0)),
            pl.BlockSpec((E, D), lambda i: (0, 0)),
        ],
        out_specs=[
            pl.BlockSpec((SB, D), lambda i: (i, 0)),
            pl.BlockSpec((SB, D), lambda i: (i, 0)),
            pl.BlockSpec((SB, 1), lambda i: (i, 0)),
            pl.BlockSpec((SB, 1), lambda i: (i, 0)),
            pl.BlockSpec((SB, 1), lambda i: (i, 0)),
            pl.BlockSpec((SB, 1), lambda i: (i, 0)),
            pl.BlockSpec((SB, 1), lambda i: (i, 0)),
            pl.BlockSpec((SB, 1), lambda i: (i, 0)),
            pl.BlockSpec((1, 16), lambda i: (0, 0)),
            pl.BlockSpec((NB, 1), lambda i: (0, 0)),
            pl.BlockSpec((1, 1), lambda i: (0, 0)),
        ],
        out_shape=(jax.ShapeDtypeStruct((S, D), F32),
                   jax.ShapeDtypeStruct((S, D), F32),
                   jax.ShapeDtypeStruct((S, 1), jnp.int32),
                   jax.ShapeDtypeStruct((S, 1), jnp.int32),
                   jax.ShapeDtypeStruct((S, 1), F32),
                   jax.ShapeDtypeStruct((S, 1), F32),
                   jax.ShapeDtypeStruct((S, 1), jnp.int32),
                   jax.ShapeDtypeStruct((S, 1), jnp.int32),
                   jax.ShapeDtypeStruct((1, 16), jnp.int32),
                   jax.ShapeDtypeStruct((NB, 1), jnp.int32),
                   jax.ShapeDtypeStruct((1, 1), jnp.int32)),
        scratch_shapes=[pltpu.VMEM((1, E), F32)],
    )(ctx2d, x2d, wo, n2, temb_p, tid, gw)


def _posw_body(i1_ref, i2_ref, r1_ref, r2_ref, w0_ref, w1_ref, st_ref,
               p1_out, p2_out, wr1_out, wr2_out):
    st8 = st_ref[...][:, :E].astype(F32)
    def pos(i_ref, r_ref, out):
        oh = (i_ref[...] == lax.broadcasted_iota(jnp.int32, (SB, E), 1)
              ).astype(F32)
        s = lax.dot_general(oh, st8, (((1,), (1,)), ((), ())),
                            precision=HI, preferred_element_type=F32)
        out[...] = s.astype(jnp.int32) + r_ref[...]
    pos(i1_ref, r1_ref, p1_out)
    pos(i2_ref, r2_ref, p2_out)
    wr1_out[...] = jnp.broadcast_to(
        w0_ref[...].astype(BF16).astype(F32), (SB, L))
    wr2_out[...] = jnp.broadcast_to(
        w1_ref[...].astype(BF16).astype(F32), (SB, L))


def _posw(i1, i2, r1, r2, w0, w1, st16):
    return pl.pallas_call(
        _posw_body,
        grid=(NSB,),
        in_specs=[
            pl.BlockSpec((SB, 1), lambda i: (i, 0)),
            pl.BlockSpec((SB, 1), lambda i: (i, 0)),
            pl.BlockSpec((SB, 1), lambda i: (i, 0)),
            pl.BlockSpec((SB, 1), lambda i: (i, 0)),
            pl.BlockSpec((SB, 1), lambda i: (i, 0)),
            pl.BlockSpec((SB, 1), lambda i: (i, 0)),
            pl.BlockSpec((1, 16), lambda i: (0, 0)),
        ],
        out_specs=[
            pl.BlockSpec((SB, 1), lambda i: (i, 0)),
            pl.BlockSpec((SB, 1), lambda i: (i, 0)),
            pl.BlockSpec((SB, L), lambda i: (i, 0)),
            pl.BlockSpec((SB, L), lambda i: (i, 0)),
        ],
        out_shape=(jax.ShapeDtypeStruct((S, 1), jnp.int32),
                   jax.ShapeDtypeStruct((S, 1), jnp.int32),
                   jax.ShapeDtypeStruct((S, L), F32),
                   jax.ShapeDtypeStruct((S, L), F32)),
    )(i1, i2, r1, r2, w0, w1, st16)


@functools.lru_cache(maxsize=None)
def _get_dispatch_sc():
    mesh = plsc.VectorSubcoreMesh(core_axis_name="c", subcore_axis_name="s")

    @functools.partial(
        pl.kernel,
        out_type=jax.ShapeDtypeStruct((P, D), F32),
        mesh=mesh,
        scratch_types=[pltpu.VMEM((TPW, D), F32),
                       pltpu.VMEM((TPW,), jnp.int32),
                       pltpu.VMEM((TPW,), jnp.int32),
                       pltpu.SemaphoreType.DMA],
    )
    def _dispatch_sc(xf_hbm, p1_hbm, p2_hbm, xs_hbm, xrows, iv1, iv2, sem):
        wid = lax.axis_index("s") * NC + lax.axis_index("c")
        base = wid * TPW
        pltpu.sync_copy(xf_hbm.at[pl.ds(base, TPW)], xrows)
        pltpu.sync_copy(p1_hbm.at[pl.ds(base, TPW)], iv1)
        pltpu.sync_copy(p2_hbm.at[pl.ds(base, TPW)], iv2)
        c1 = pltpu.async_copy(xrows, xs_hbm.at[iv1], sem)
        c2 = pltpu.async_copy(xrows, xs_hbm.at[iv2], sem)
        c1.wait()
        c2.wait()

    return _dispatch_sc


def _ffn1_body(be_ref, nb_ref, xs_ref, w1_ref, b1_ref, h1_out):
    b = pl.program_id(0)

    @pl.when(b < nb_ref[0])
    def _():
        h1 = lax.dot_general(xs_ref[...], w1_ref[0], (((1,), (1,)), ((), ())),
                             preferred_element_type=F32)
        h1_out[...] = (h1 + b1_ref[0]).astype(BF16)


def _ffn1(be, nb, xs, w1b, b1):
    return pl.pallas_call(
        _ffn1_body,
        grid_spec=pltpu.PrefetchScalarGridSpec(
            num_scalar_prefetch=2,
            grid=(NB,),
            in_specs=[
                pl.BlockSpec((BLK, D),
                             lambda b, be, nb: (jnp.minimum(b, nb[0] - 1), 0)),
                pl.BlockSpec((1, HID, D), lambda b, be, nb: (be[b], 0, 0)),
                pl.BlockSpec((1, 1, HID), lambda b, be, nb: (be[b], 0, 0)),
            ],
            out_specs=pl.BlockSpec(
                (BLK, HID), lambda b, be, nb: (jnp.minimum(b, nb[0] - 1), 0)),
        ),
        out_shape=jax.ShapeDtypeStruct((P, HID), BF16),
    )(be, nb, xs, w1b, b1)


def _ffn2_body(be_ref, nb_ref, h1_ref, wg_ref, wv_ref, bg_ref, bv_ref,
               ys_out):
    b = pl.program_id(0)

    @pl.when(b < nb_ref[0])
    def _():
        h1 = h1_ref[...]
        go = lax.dot_general(h1, wg_ref[0], (((1,), (1,)), ((), ())),
                             preferred_element_type=F32) + bg_ref[0]
        vo = lax.dot_general(h1, wv_ref[0], (((1,), (1,)), ((), ())),
                             preferred_element_type=F32) + bv_ref[0]
        eo = go * (1.0 / (1.0 + jnp.exp(-go))) * vo
        ys_out[...] = eo.astype(BF16).astype(F32)


def _ffn2(be, nb, h1, wgb, wvb, bg, bv):
    return pl.pallas_call(
        _ffn2_body,
        grid_spec=pltpu.PrefetchScalarGridSpec(
            num_scalar_prefetch=2,
            grid=(NB,),
            in_specs=[
                pl.BlockSpec((BLK, HID),
                             lambda b, be, nb: (jnp.minimum(b, nb[0] - 1), 0)),
                pl.BlockSpec((1, D, HID), lambda b, be, nb: (be[b], 0, 0)),
                pl.BlockSpec((1, D, HID), lambda b, be, nb: (be[b], 0, 0)),
                pl.BlockSpec((1, 1, D), lambda b, be, nb: (be[b], 0, 0)),
                pl.BlockSpec((1, 1, D), lambda b, be, nb: (be[b], 0, 0)),
            ],
            out_specs=pl.BlockSpec(
                (BLK, D), lambda b, be, nb: (jnp.minimum(b, nb[0] - 1), 0)),
        ),
        out_shape=jax.ShapeDtypeStruct((P, D), F32),
    )(be, nb, h1, wgb, wvb, bg, bv)


@functools.lru_cache(maxsize=None)
def _get_combine_sc():
    mesh = plsc.VectorSubcoreMesh(core_axis_name="c", subcore_axis_name="s")

    @functools.partial(
        pl.kernel,
        out_type=jax.ShapeDtypeStruct((S, D), F32),
        mesh=mesh,
        scratch_types=[pltpu.VMEM((HROWS, D), F32),
                       pltpu.VMEM((HROWS, D), F32),
                       pltpu.VMEM((HROWS, D), F32),
                       pltpu.VMEM((HROWS,), jnp.int32),
                       pltpu.VMEM((HROWS,), jnp.int32),
                       pltpu.VMEM((HROWS, L), F32),
                       pltpu.VMEM((HROWS, L), F32),
                       pltpu.SemaphoreType.DMA],
    )
    def _combine_sc(x2_hbm, ys_hbm, pos1_hbm, pos2_hbm, wr1_hbm, wr2_hbm,
                    out_hbm, xr, y1, y2, p1, p2, wv1, wv2, sem):
        wid = lax.axis_index("s") * NC + lax.axis_index("c")
        for half in range(TPW // HROWS):
            base = wid * TPW + half * HROWS
            pltpu.sync_copy(x2_hbm.at[pl.ds(base, HROWS)], xr)
            pltpu.sync_copy(pos1_hbm.at[pl.ds(base, HROWS)], p1)
            pltpu.sync_copy(pos2_hbm.at[pl.ds(base, HROWS)], p2)
            pltpu.sync_copy(wr1_hbm.at[pl.ds(base, HROWS)], wv1)
            pltpu.sync_copy(wr2_hbm.at[pl.ds(base, HROWS)], wv2)
            c1 = pltpu.async_copy(ys_hbm.at[p1], y1, sem)
            c2 = pltpu.async_copy(ys_hbm.at[p2], y2, sem)
            c1.wait()
            c2.wait()
            for j in range(HROWS):
                xrj = xr.at[j]
                y1j = y1.at[j]
                y2j = y2.at[j]
                w0v = wv1[j]
                w1v = wv2[j]

                def body(c, carry, xrj=xrj, y1j=y1j, y2j=y2j,
                         w0v=w0v, w1v=w1v):
                    sl = pl.ds(c * L, L)
                    xrj[sl] = xrj[sl] + w0v * y1j[sl] + w1v * y2j[sl]
                    return carry

                lax.fori_loop(0, D // L, body, 0)
            pltpu.sync_copy(xr, out_hbm.at[pl.ds(base, HROWS)])

    return _combine_sc


def kernel(x, freqs_cos, freqs_sin, task_ids, n1w, n2w, Wq, Wk, Wv, Wo,
           qA, qB, kA, kB, vA, vB, gateW, gA, gB, temb, W1, b1, Wg, bg,
           Wve, bv):
    x2d = x.reshape(S, D)
    cos = freqs_cos.reshape(S, DK)
    sin = freqs_sin.reshape(S, DK)
    tid = task_ids.reshape(S, 1).astype(jnp.int32)

    wq, wk, wv, gw, wo = _merge(Wq, Wk, Wv, gateW, Wo,
                                qA, qB, kA, kB, vA, vB, gA, gB)
    q2d, kr, vr = _qkv(x2d, cos, sin, n1w.reshape(1, D), wq, wk, wv)
    ctx_a = _attn_part(q2d, kr, vr, 0, NSB // 2, S // 2)
    ctx_b = _attn_part(q2d, kr, vr, NSB // 2, NSB // 2, S)
    ctx2d = jnp.concatenate([ctx_a, ctx_b], axis=0)
    temb_p = jnp.pad(temb, ((0, E - NT), (0, 0)))
    (x2, xf, i1, i2, w0, w1, r1, r2, st16, be, nb) = _post(
        ctx2d, x2d, wo, n2w.reshape(1, D), temb_p, tid, gw)
    pos1, pos2, wr1, wr2 = _posw(i1, i2, r1, r2, w0, w1, st16)
    xs = _get_dispatch_sc()(xf, pos1.reshape(S), pos2.reshape(S))
    h1 = _ffn1(be.reshape(NB), nb.reshape(1), xs, W1,
               b1.reshape(E, 1, HID))
    ys = _ffn2(be.reshape(NB), nb.reshape(1), h1, Wg.astype(BF16),
               Wve.astype(BF16), bg.reshape(E, 1, D), bv.reshape(E, 1, D))
    out = _get_combine_sc()(x2, ys, pos1.reshape(S), pos2.reshape(S),
                            wr1, wr2)
    return out.reshape(1, S, D)


# speedup vs baseline: 1.6625x; 1.0455x over previous
import functools

import jax
import jax.numpy as jnp
import numpy as np
from jax import lax
from jax.experimental import pallas as pl
from jax.experimental.pallas import tpu as pltpu
from jax.experimental.pallas import tpu_sc as plsc

S, D = 2048, 1024
NH, NKV, DK = 16, 4, 64
E, TOPK, R, NT = 8, 2, 16, 3
HID = D * 4
KVD = NKV * DK
SCALING = 2.0
GQ = NH // NKV
GD = GQ * DK
HALF = DK // 2
SB = 256
NSB = S // SB
BLK = 256
NB = (S * TOPK) // BLK + E
P = NB * BLK
NEG = float(np.finfo(np.float32).min)
F32 = jnp.float32
BF16 = jnp.bfloat16
HI = jax.lax.Precision.HIGHEST

NC, NS, L = 2, 16, 16
NW = NC * NS
TPW = S // NW
HROWS = 32


def _merge_body(wq_ref, wk_ref, wv_ref, gw_ref, wo_ref, qa_ref, qb_ref,
                ka_ref, kb_ref, va_ref, vb_ref, ga_ref, gb_ref,
                oq, ok, ov, og, oo):
    def m(w0, a, bm, out, dt):
        up = lax.dot_general(bm[...], a[...], (((0,), (1,)), ((), ())),
                             preferred_element_type=F32)
        out[...] = (w0[...] + SCALING * up).astype(dt)
    m(wq_ref, qa_ref, qb_ref, oq, BF16)
    m(wk_ref, ka_ref, kb_ref, ok, BF16)
    m(wv_ref, va_ref, vb_ref, ov, BF16)
    m(gw_ref, ga_ref, gb_ref, og, F32)
    oo[...] = wo_ref[...].astype(BF16)


def _merge(Wq, Wk, Wv, gateW, Wo, qA, qB, kA, kB, vA, vB, gA, gB):
    return pl.pallas_call(
        _merge_body,
        out_shape=(jax.ShapeDtypeStruct((D, D), BF16),
                   jax.ShapeDtypeStruct((KVD, D), BF16),
                   jax.ShapeDtypeStruct((KVD, D), BF16),
                   jax.ShapeDtypeStruct((E, D), F32),
                   jax.ShapeDtypeStruct((D, D), BF16)),
    )(Wq, Wk, Wv, gateW, Wo, qA, qB, kA, kB, vA, vB, gA, gB)


def _qkv_body(x_ref, cos_ref, sin_ref, n1_ref, wq_ref, wk_ref, wv_ref,
              q_out, k_out, v_out):
    xb = x_ref[...]
    ms = jnp.mean(xb * xb, axis=-1, keepdims=True)
    h = (xb * lax.rsqrt(ms + 1e-6) * n1_ref[...]).astype(BF16)
    q = lax.dot_general(h, wq_ref[...], (((1,), (1,)), ((), ())),
                        preferred_element_type=F32)
    k = lax.dot_general(h, wk_ref[...], (((1,), (1,)), ((), ())),
                        preferred_element_type=F32)
    v = lax.dot_general(h, wv_ref[...], (((1,), (1,)), ((), ())),
                        preferred_element_type=F32)
    cos = cos_ref[...]
    sin = sin_ref[...]
    c1, c2 = cos[:, :HALF], cos[:, HALF:]
    s1, s2 = sin[:, :HALF], sin[:, HALF:]
    pieces = []
    for hh in range(NH):
        qh = q[:, hh * DK:(hh + 1) * DK]
        q1, q2 = qh[:, :HALF], qh[:, HALF:]
        pieces.append(jnp.concatenate(
            [q1 * c1 - q2 * s1, q2 * c2 + q1 * s2], axis=-1))
    q_out[...] = jnp.concatenate(pieces, axis=-1).astype(BF16)
    for hh in range(NKV):
        kh = k[:, hh * DK:(hh + 1) * DK]
        k1, k2 = kh[:, :HALF], kh[:, HALF:]
        k_out[hh] = jnp.concatenate(
            [k1 * c1 - k2 * s1, k2 * c2 + k1 * s2], axis=-1).astype(BF16)
        v_out[hh] = v[:, hh * DK:(hh + 1) * DK].astype(BF16)


def _qkv(x2d, cos, sin, n1, wq, wk, wv):
    return pl.pallas_call(
        _qkv_body,
        grid=(NSB,),
        in_specs=[
            pl.BlockSpec((SB, D), lambda i: (i, 0)),
            pl.BlockSpec((SB, DK), lambda i: (i, 0)),
            pl.BlockSpec((SB, DK), lambda i: (i, 0)),
            pl.BlockSpec((1, D), lambda i: (0, 0)),
            pl.BlockSpec((D, D), lambda i: (0, 0)),
            pl.BlockSpec((KVD, D), lambda i: (0, 0)),
            pl.BlockSpec((KVD, D), lambda i: (0, 0)),
        ],
        out_specs=[
            pl.BlockSpec((SB, D), lambda i: (i, 0)),
            pl.BlockSpec((NKV, SB, DK), lambda i: (0, i, 0)),
            pl.BlockSpec((NKV, SB, DK), lambda i: (0, i, 0)),
        ],
        out_shape=(jax.ShapeDtypeStruct((S, D), BF16),
                   jax.ShapeDtypeStruct((NKV, S, DK), BF16),
                   jax.ShapeDtypeStruct((NKV, S, DK), BF16)),
    )(x2d, cos, sin, n1, wq, wk, wv)


def _attn_body(q_ref, k_ref, v_ref, o_ref, *, q0, kw):
    i = pl.program_id(1)
    row = (q0 + i) * SB + lax.broadcasted_iota(jnp.int32, (SB, kw), 0)
    col = lax.broadcasted_iota(jnp.int32, (SB, kw), 1)
    causal = col <= row
    qg = q_ref[...] * BF16(0.125)
    k = k_ref[0]
    v = v_ref[0]
    pieces = []
    for hh in range(GQ):
        q = qg[:, hh * DK:(hh + 1) * DK]
        s = lax.dot_general(q, k, (((1,), (1,)), ((), ())),
                            preferred_element_type=F32)
        p = jnp.where(causal, jnp.exp(s), 0.0)
        lsum = jnp.sum(p, axis=-1, keepdims=True)
        attn = (p * (1.0 / lsum)).astype(BF16)
        pieces.append(jnp.dot(attn, v, preferred_element_type=F32))
    o_ref[...] = jnp.concatenate(pieces, axis=-1).astype(BF16)


def _attn_part(q2d, kr, vr, q0, nqb, kw):
    body = functools.partial(_attn_body, q0=q0, kw=kw)
    return pl.pallas_call(
        body,
        grid=(NKV, nqb),
        in_specs=[
            pl.BlockSpec((SB, GD), lambda g, i: (q0 + i, g)),
            pl.BlockSpec((1, kw, DK), lambda g, i: (g, 0, 0)),
            pl.BlockSpec((1, kw, DK), lambda g, i: (g, 0, 0)),
        ],
        out_specs=pl.BlockSpec((SB, GD), lambda g, i: (i, g)),
        out_shape=jax.ShapeDtypeStruct((nqb * SB, D), BF16),
    )(q2d, kr, vr)


def _post_body(ctx_ref, x_ref, wo_ref, n2_ref, temb_ref, tid_ref, gw_ref,
               x2_out, xf_out, i1_out, i2_out, w0_out, w1_out,
               r1_out, r2_out, st_out, be_out, nb_out, base):
    step = pl.program_id(0)
    x2 = x_ref[...] + lax.dot_general(ctx_ref[...], wo_ref[...],
                                      (((1,), (1,)), ((), ())),
                                      preferred_element_type=F32)
    x2_out[...] = x2
    ms = jnp.mean(x2 * x2, axis=-1, keepdims=True)
    h2 = x2 * lax.rsqrt(ms + 1e-6) * n2_ref[...]
    tid = tid_ref[...]
    toh = (tid == lax.broadcasted_iota(jnp.int32, (SB, E), 1)).astype(F32)
    xf = h2 + lax.dot_general(toh, temb_ref[...], (((1,), (0,)), ((), ())),
                              precision=HI, preferred_element_type=F32)
    xf_out[...] = xf
    gl = lax.dot_general(xf, gw_ref[...], (((1,), (1,)), ((), ())),
                         preferred_element_type=F32)
    e_iota = lax.broadcasted_iota(jnp.int32, (SB, E), 1)
    m1 = jnp.max(gl, axis=-1, keepdims=True)
    i1 = jnp.min(jnp.where(gl == m1, e_iota, E), axis=-1, keepdims=True)
    glm = jnp.where(e_iota == i1, NEG, gl)
    m2 = jnp.max(glm, axis=-1, keepdims=True)
    i2 = jnp.min(jnp.where(glm == m2, e_iota, E), axis=-1, keepdims=True)
    ev = jnp.exp(m2 - m1)
    w0_out[...] = 1.0 / (1.0 + ev)
    w1_out[...] = ev / (1.0 + ev)
    i1_out[...] = i1.astype(jnp.int32)
    i2_out[...] = i2.astype(jnp.int32)

    @pl.when(step == 0)
    def _():
        base[...] = jnp.zeros((1, E), F32)

    pair = jnp.concatenate([i1, i2], axis=0).astype(jnp.int32)
    oh = (pair == lax.broadcasted_iota(jnp.int32, (2 * SB, E), 1)).astype(F32)
    ri = lax.broadcasted_iota(jnp.int32, (2 * SB, 2 * SB), 0)
    ci = lax.broadcasted_iota(jnp.int32, (2 * SB, 2 * SB), 1)
    lstrict = (ci < ri).astype(F32)
    prior = lax.dot_general(lstrict, oh, (((1,), (0,)), ((), ())),
                            precision=HI, preferred_element_type=F32)
    rank_all = prior + base[...]
    r = jnp.sum(rank_all * oh, axis=-1, keepdims=True).astype(jnp.int32)
    r1_out[...] = r[:SB]
    r2_out[...] = r[SB:]
    base[...] = base[...] + jnp.sum(oh, axis=0, keepdims=True)

    @pl.when(step == NSB - 1)
    def _():
        cntf = base[...]
        nbe = jnp.floor((cntf + (BLK - 1)) / BLK)
        li = lax.broadcasted_iota(jnp.int32, (E, E), 0)
        lj = lax.broadcasted_iota(jnp.int32, (E, E), 1)
        mstrict = (li < lj).astype(F32)
        sbk = lax.dot_general(nbe, mstrict, (((1,), (0,)), ((), ())),
                              precision=HI, preferred_element_type=F32)
        st_out[...] = jnp.concatenate(
            [sbk * BLK, jnp.zeros((1, E), F32)], axis=1).astype(jnp.int32)
        nbt = jnp.sum(nbe, axis=-1, keepdims=True)
        nb_out[...] = nbt.astype(jnp.int32)
        bio = lax.broadcasted_iota(jnp.int32, (NB, E), 0).astype(F32)
        bcl = jnp.minimum(bio, jnp.broadcast_to(nbt, (NB, E)) - 1.0)
        sbk_b = jnp.broadcast_to(sbk, (NB, E))
        be = jnp.sum((sbk_b <= bcl).astype(F32), axis=-1, keepdims=True) - 1.0
        be_out[...] = be.astype(jnp.int32)


def _post(ctx2d, x2d, wo, n2, temb_p, tid, gw):
    return pl.pallas_call(
        _post_body,
        grid=(NSB,),
        in_specs=[
            pl.BlockSpec((SB, D), lambda i: (i, 0)),
            pl.BlockSpec((SB, D), lambda i: (i, 0)),
            pl.BlockSpec((D, D), lambda i: (0, 0)),
            pl.BlockSpec((1, D), lambda i: (0, 0)),
            pl.BlockSpec((E, D), lambda i: (0, 0)),
            pl.BlockSpec((SB, 1), lambda i: (i, 0)),
            pl.BlockSpec((E, D), lambda i: (0, 0)),
        ],
        out_specs=[
            pl.BlockSpec((SB, D), lambda i: (i, 0)),
            pl.BlockSpec((SB, D), lambda i: (i, 0)),
            pl.BlockSpec((SB, 1), lambda i: (i, 0)),
            pl.BlockSpec((SB, 1), lambda i: (i, 0)),
            pl.BlockSpec((SB, 1), lambda i: (i, 0)),
            pl.BlockSpec((SB, 1), lambda i: (i, 0)),
            pl.BlockSpec((SB, 1), lambda i: (i, 0)),
            pl.BlockSpec((SB, 1), lambda i: (i, 0)),
            pl.BlockSpec((1, 16), lambda i: (0, 0)),
            pl.BlockSpec((NB, 1), lambda i: (0, 0)),
            pl.BlockSpec((1, 1), lambda i: (0, 0)),
        ],
        out_shape=(jax.ShapeDtypeStruct((S, D), F32),
                   jax.ShapeDtypeStruct((S, D), F32),
                   jax.ShapeDtypeStruct((S, 1), jnp.int32),
                   jax.ShapeDtypeStruct((S, 1), jnp.int32),
                   jax.ShapeDtypeStruct((S, 1), F32),
                   jax.ShapeDtypeStruct((S, 1), F32),
                   jax.ShapeDtypeStruct((S, 1), jnp.int32),
                   jax.ShapeDtypeStruct((S, 1), jnp.int32),
                   jax.ShapeDtypeStruct((1, 16), jnp.int32),
                   jax.ShapeDtypeStruct((NB, 1), jnp.int32),
                   jax.ShapeDtypeStruct((1, 1), jnp.int32)),
        scratch_shapes=[pltpu.VMEM((1, E), F32)],
    )(ctx2d, x2d, wo, n2, temb_p, tid, gw)


def _posw_body(i1_ref, i2_ref, r1_ref, r2_ref, w0_ref, w1_ref, st_ref,
               p1_out, p2_out, wr1_out, wr2_out):
    st8 = st_ref[...][:, :E].astype(F32)
    def pos(i_ref, r_ref, out):
        oh = (i_ref[...] == lax.broadcasted_iota(jnp.int32, (SB, E), 1)
              ).astype(F32)
        s = lax.dot_general(oh, st8, (((1,), (1,)), ((), ())),
                            precision=HI, preferred_element_type=F32)
        out[...] = s.astype(jnp.int32) + r_ref[...]
    pos(i1_ref, r1_ref, p1_out)
    pos(i2_ref, r2_ref, p2_out)
    wr1_out[...] = jnp.broadcast_to(
        w0_ref[...].astype(BF16).astype(F32), (SB, L))
    wr2_out[...] = jnp.broadcast_to(
        w1_ref[...].astype(BF16).astype(F32), (SB, L))


def _posw(i1, i2, r1, r2, w0, w1, st16):
    return pl.pallas_call(
        _posw_body,
        grid=(NSB,),
        in_specs=[
            pl.BlockSpec((SB, 1), lambda i: (i, 0)),
            pl.BlockSpec((SB, 1), lambda i: (i, 0)),
            pl.BlockSpec((SB, 1), lambda i: (i, 0)),
            pl.BlockSpec((SB, 1), lambda i: (i, 0)),
            pl.BlockSpec((SB, 1), lambda i: (i, 0)),
            pl.BlockSpec((SB, 1), lambda i: (i, 0)),
            pl.BlockSpec((1, 16), lambda i: (0, 0)),
        ],
        out_specs=[
            pl.BlockSpec((SB, 1), lambda i: (i, 0)),
            pl.BlockSpec((SB, 1), lambda i: (i, 0)),
            pl.BlockSpec((SB, L), lambda i: (i, 0)),
            pl.BlockSpec((SB, L), lambda i: (i, 0)),
        ],
        out_shape=(jax.ShapeDtypeStruct((S, 1), jnp.int32),
                   jax.ShapeDtypeStruct((S, 1), jnp.int32),
                   jax.ShapeDtypeStruct((S, L), F32),
                   jax.ShapeDtypeStruct((S, L), F32)),
    )(i1, i2, r1, r2, w0, w1, st16)


@functools.lru_cache(maxsize=None)
def _get_dispatch_sc():
    mesh = plsc.VectorSubcoreMesh(core_axis_name="c", subcore_axis_name="s")

    @functools.partial(
        pl.kernel,
        out_type=jax.ShapeDtypeStruct((P, D), F32),
        mesh=mesh,
        scratch_types=[pltpu.VMEM((TPW, D), F32),
                       pltpu.VMEM((TPW,), jnp.int32),
                       pltpu.VMEM((TPW,), jnp.int32),
                       pltpu.SemaphoreType.DMA],
    )
    def _dispatch_sc(xf_hbm, p1_hbm, p2_hbm, xs_hbm, xrows, iv1, iv2, sem):
        wid = lax.axis_index("s") * NC + lax.axis_index("c")
        base = wid * TPW
        pltpu.sync_copy(xf_hbm.at[pl.ds(base, TPW)], xrows)
        pltpu.sync_copy(p1_hbm.at[pl.ds(base, TPW)], iv1)
        pltpu.sync_copy(p2_hbm.at[pl.ds(base, TPW)], iv2)
        c1 = pltpu.async_copy(xrows, xs_hbm.at[iv1], sem)
        c2 = pltpu.async_copy(xrows, xs_hbm.at[iv2], sem)
        c1.wait()
        c2.wait()

    return _dispatch_sc


def _ffn1_body(be_ref, nb_ref, xs_ref, w1_ref, b1_ref, h1_out):
    b = pl.program_id(0)

    @pl.when(b < nb_ref[0])
    def _():
        h1 = lax.dot_general(xs_ref[...], w1_ref[0], (((1,), (1,)), ((), ())),
                             preferred_element_type=F32)
        h1_out[...] = (h1 + b1_ref[0]).astype(BF16)


def _ffn1(be, nb, xs, w1b, b1):
    return pl.pallas_call(
        _ffn1_body,
        grid_spec=pltpu.PrefetchScalarGridSpec(
            num_scalar_prefetch=2,
            grid=(NB,),
            in_specs=[
                pl.BlockSpec((BLK, D),
                             lambda b, be, nb: (jnp.minimum(b, nb[0] - 1), 0)),
                pl.BlockSpec((1, HID, D), lambda b, be, nb: (be[b], 0, 0)),
                pl.BlockSpec((1, 1, HID), lambda b, be, nb: (be[b], 0, 0)),
            ],
            out_specs=pl.BlockSpec(
                (BLK, HID), lambda b, be, nb: (jnp.minimum(b, nb[0] - 1), 0)),
        ),
        out_shape=jax.ShapeDtypeStruct((P, HID), BF16),
    )(be, nb, xs, w1b, b1)


def _ffn2_body(be_ref, nb_ref, h1_ref, wg_ref, wv_ref, bg_ref, bv_ref,
               ys_out):
    b = pl.program_id(0)

    @pl.when(b < nb_ref[0])
    def _():
        h1 = h1_ref[...]
        go = lax.dot_general(h1, wg_ref[0], (((1,), (1,)), ((), ())),
                             preferred_element_type=F32) + bg_ref[0]
        vo = lax.dot_general(h1, wv_ref[0], (((1,), (1,)), ((), ())),
                             preferred_element_type=F32) + bv_ref[0]
        eo = go * (1.0 / (1.0 + jnp.exp(-go))) * vo
        ys_out[...] = eo.astype(BF16).astype(F32)


def _ffn2(be, nb, h1, wgb, wvb, bg, bv):
    return pl.pallas_call(
        _ffn2_body,
        grid_spec=pltpu.PrefetchScalarGridSpec(
            num_scalar_prefetch=2,
            grid=(NB,),
            in_specs=[
                pl.BlockSpec((BLK, HID),
                             lambda b, be, nb: (jnp.minimum(b, nb[0] - 1), 0)),
                pl.BlockSpec((1, D, HID), lambda b, be, nb: (be[b], 0, 0)),
                pl.BlockSpec((1, D, HID), lambda b, be, nb: (be[b], 0, 0)),
                pl.BlockSpec((1, 1, D), lambda b, be, nb: (be[b], 0, 0)),
                pl.BlockSpec((1, 1, D), lambda b, be, nb: (be[b], 0, 0)),
            ],
            out_specs=pl.BlockSpec(
                (BLK, D), lambda b, be, nb: (jnp.minimum(b, nb[0] - 1), 0)),
        ),
        out_shape=jax.ShapeDtypeStruct((P, D), F32),
    )(be, nb, h1, wgb, wvb, bg, bv)


@functools.lru_cache(maxsize=None)
def _get_combine_sc():
    mesh = plsc.VectorSubcoreMesh(core_axis_name="c", subcore_axis_name="s")

    @functools.partial(
        pl.kernel,
        out_type=jax.ShapeDtypeStruct((S, D), F32),
        mesh=mesh,
        scratch_types=[pltpu.VMEM((HROWS, D), F32),
                       pltpu.VMEM((HROWS, D), F32),
                       pltpu.VMEM((HROWS, D), F32),
                       pltpu.VMEM((HROWS,), jnp.int32),
                       pltpu.VMEM((HROWS,), jnp.int32),
                       pltpu.VMEM((HROWS, L), F32),
                       pltpu.VMEM((HROWS, L), F32),
                       pltpu.SemaphoreType.DMA],
    )
    def _combine_sc(x2_hbm, ys_hbm, pos1_hbm, pos2_hbm, wr1_hbm, wr2_hbm,
                    out_hbm, xr, y1, y2, p1, p2, wv1, wv2, sem):
        wid = lax.axis_index("s") * NC + lax.axis_index("c")
        for half in range(TPW // HROWS):
            base = wid * TPW + half * HROWS
            pltpu.sync_copy(x2_hbm.at[pl.ds(base, HROWS)], xr)
            pltpu.sync_copy(pos1_hbm.at[pl.ds(base, HROWS)], p1)
            pltpu.sync_copy(pos2_hbm.at[pl.ds(base, HROWS)], p2)
            pltpu.sync_copy(wr1_hbm.at[pl.ds(base, HROWS)], wv1)
            pltpu.sync_copy(wr2_hbm.at[pl.ds(base, HROWS)], wv2)
            c1 = pltpu.async_copy(ys_hbm.at[p1], y1, sem)
            c2 = pltpu.async_copy(ys_hbm.at[p2], y2, sem)
            c1.wait()
            c2.wait()
            for j in range(HROWS):
                xrj = xr.at[j]
                y1j = y1.at[j]
                y2j = y2.at[j]
                w0v = wv1[j]
                w1v = wv2[j]

                def body(c, carry, xrj=xrj, y1j=y1j, y2j=y2j,
                         w0v=w0v, w1v=w1v):
                    sl = pl.ds(c * L, L)
                    xrj[sl] = xrj[sl] + w0v * y1j[sl] + w1v * y2j[sl]
                    return carry

                lax.fori_loop(0, D // L, body, 0)
            pltpu.sync_copy(xr, out_hbm.at[pl.ds(base, HROWS)])

    return _combine_sc


def kernel(x, freqs_cos, freqs_sin, task_ids, n1w, n2w, Wq, Wk, Wv, Wo,
           qA, qB, kA, kB, vA, vB, gateW, gA, gB, temb, W1, b1, Wg, bg,
           Wve, bv):
    x2d = x.reshape(S, D)
    cos = freqs_cos.reshape(S, DK)
    sin = freqs_sin.reshape(S, DK)
    tid = task_ids.reshape(S, 1).astype(jnp.int32)

    wq, wk, wv, gw, wo = _merge(Wq, Wk, Wv, gateW, Wo,
                                qA, qB, kA, kB, vA, vB, gA, gB)
    q2d, kr, vr = _qkv(x2d, cos, sin, n1w.reshape(1, D), wq, wk, wv)
    ctx_a = _attn_part(q2d, kr, vr, 0, NSB // 2, S // 2)
    ctx_b = _attn_part(q2d, kr, vr, NSB // 2, NSB // 2, S)
    ctx2d = jnp.concatenate([ctx_a, ctx_b], axis=0)
    temb_p = jnp.pad(temb, ((0, E - NT), (0, 0)))
    (x2, xf, i1, i2, w0, w1, r1, r2, st16, be, nb) = _post(
        ctx2d, x2d, wo, n2w.reshape(1, D), temb_p, tid, gw)
    pos1, pos2, wr1, wr2 = _posw(i1, i2, r1, r2, w0, w1, st16)
    xs = _get_dispatch_sc()(xf, pos1.reshape(S), pos2.reshape(S))
    h1 = _ffn1(be.reshape(NB), nb.reshape(1), xs, W1,
               b1.reshape(E, 1, HID))
    ys = _ffn2(be.reshape(NB), nb.reshape(1), h1, Wg.astype(BF16),
               Wve.astype(BF16), bg.reshape(E, 1, D), bv.reshape(E, 1, D))
    out = _get_combine_sc()(x2, ys, pos1.reshape(S), pos2.reshape(S),
                            wr1, wr2)
    return out.reshape(1, S, D)
